# Initial kernel scaffold; baseline (speedup 1.0000x reference)
import jax
import jax.numpy as jnp
from jax.experimental import pallas as pl


def kernel(edge_index, x, Wp, bp, W1, b1, W2, b2):
    raise NotImplementedError("write your pallas kernel here")


# speedup vs baseline: 9.8151x; 9.8151x over previous
import functools

import jax
import jax.numpy as jnp
from jax import lax
from jax.experimental import pallas as pl
from jax.experimental.pallas import tpu as pltpu
from jax.experimental.pallas import tpu_sc as plsc

N = 10000
E = 160000
F = 256
HALF = 128
NC, NS = 2, 16
NPAD = 10112
RPT = NPAD // NS
CHUNK = 128
NCHUNKS = E // CHUNK


def _sc_mesh():
    return plsc.VectorSubcoreMesh(core_axis_name="c", subcore_axis_name="s")


def _deg(edge_index, ones_v, zeros_half):
    @functools.partial(
        pl.kernel,
        out_type=jax.ShapeDtypeStruct((NC, NPAD, HALF), jnp.float32),
        mesh=_sc_mesh(),
        scratch_types=[
            pltpu.VMEM((CHUNK, HALF), jnp.float32),
            pltpu.VMEM((2, CHUNK), jnp.int32),
            pltpu.VMEM_SHARED((NPAD, HALF), jnp.float32),
            pltpu.SemaphoreType.DMA,
        ],
    )
    def k(edge_hbm, ones_hbm, zeros_hbm, deg_hbm, ones_vm, eidx_vm, acc, sem):
        c = lax.axis_index("c")
        t = lax.axis_index("s")
        pltpu.sync_copy(ones_hbm, ones_vm)
        pltpu.sync_copy(zeros_hbm, acc.at[pl.ds(t * RPT, RPT)])
        plsc.subcore_barrier()

        @pl.loop(c * (NCHUNKS // NC) + t, (c + 1) * (NCHUNKS // NC), step=NS)
        def _(i):
            pltpu.sync_copy(edge_hbm.at[:, pl.ds(i * CHUNK, CHUNK)], eidx_vm)
            pltpu.sync_copy(ones_vm, acc.at[eidx_vm.at[1]], add=True)

        plsc.subcore_barrier()
        pltpu.sync_copy(acc.at[pl.ds(t * RPT, RPT)],
                        deg_hbm.at[c, pl.ds(t * RPT, RPT)])

    return k(edge_index, ones_v, zeros_half)


def _agg(edge_index, g2n, zeros_half):
    @functools.partial(
        pl.kernel,
        out_type=jax.ShapeDtypeStruct((NC, NPAD, HALF), jnp.float32),
        mesh=_sc_mesh(),
        scratch_types=[
            pltpu.VMEM((2, CHUNK), jnp.int32),
            pltpu.VMEM((CHUNK,), jnp.int32),
            pltpu.VMEM((CHUNK, HALF), jnp.float32),
            pltpu.VMEM_SHARED((NPAD, HALF), jnp.float32),
            pltpu.SemaphoreType.DMA,
        ],
    )
    def k(edge_hbm, g_hbm, zeros_hbm, s_hbm, eidx_vm, gidx_vm, rows_vm, acc, sem):
        c = lax.axis_index("c")
        t = lax.axis_index("s")
        base = c * N
        pltpu.sync_copy(zeros_hbm, acc.at[pl.ds(t * RPT, RPT)])
        plsc.subcore_barrier()

        @pl.loop(t, NCHUNKS, step=NS)
        def _(i):
            pltpu.sync_copy(edge_hbm.at[:, pl.ds(i * CHUNK, CHUNK)], eidx_vm)
            for v in range(CHUNK // 16):
                sl = pl.ds(v * 16, 16)
                gidx_vm[sl] = eidx_vm[0, sl] + base
            pltpu.async_copy(g_hbm.at[gidx_vm], rows_vm, sem).wait()
            pltpu.sync_copy(rows_vm, acc.at[eidx_vm.at[1]], add=True)

        plsc.subcore_barrier()
        pltpu.sync_copy(acc.at[pl.ds(t * RPT, RPT)],
                        s_hbm.at[c, pl.ds(t * RPT, RPT)])

    return k(edge_index, g2n, zeros_half)


_PREC = lax.Precision.HIGHEST


def _dinv_of(degp_blk):
    deg = degp_blk[0, :, 0] + degp_blk[1, :, 0] + 1.0
    return lax.rsqrt(deg)


def _mm1_body(x_ref, w1_ref, wp_ref, bp_ref, degp_ref, g1_ref, xp_ref):
    dinv = _dinv_of(degp_ref[...])
    m = jnp.dot(x_ref[...], w1_ref[...], preferred_element_type=jnp.float32,
                precision=_PREC)
    g1_ref[0] = dinv[:, None] * m
    xp_ref[...] = jnp.dot(x_ref[...], wp_ref[...],
                          preferred_element_type=jnp.float32,
                          precision=_PREC) + bp_ref[0]


def _mm1(x, W1, Wp_pad, bp_pad, degp, blk):
    grid = (N // blk, 2)
    return pl.pallas_call(
        _mm1_body,
        grid=grid,
        in_specs=[
            pl.BlockSpec((blk, F), lambda i, j: (i, 0)),
            pl.BlockSpec((F, HALF), lambda i, j: (0, j)),
            pl.BlockSpec((F, HALF), lambda i, j: (0, 0)),
            pl.BlockSpec((1, HALF), lambda i, j: (0, 0)),
            pl.BlockSpec((2, blk, HALF), lambda i, j: (0, i, 0)),
        ],
        out_specs=[
            pl.BlockSpec((1, blk, HALF), lambda i, j: (j, i, 0)),
            pl.BlockSpec((blk, HALF), lambda i, j: (i, 0)),
        ],
        out_shape=[
            jax.ShapeDtypeStruct((2, N, HALF), jnp.float32),
            jax.ShapeDtypeStruct((N, HALF), jnp.float32),
        ],
    )(x, W1, Wp_pad, bp_pad, degp)


def _mm2_body(s1_ref, g1_ref, degp_ref, b1_ref, xp_ref, w2a_ref, w2b_ref,
              g2_ref):
    dinv = _dinv_of(degp_ref[...])
    pre = dinv[None, :, None] * (s1_ref[...] + g1_ref[...])
    h = jnp.concatenate([pre[0], pre[1]], axis=1) + b1_ref[0]
    h = jnp.maximum(h, 0.0)
    h2 = (jnp.dot(h, w2a_ref[...], preferred_element_type=jnp.float32,
                  precision=_PREC)
          + jnp.dot(xp_ref[...], w2b_ref[...],
                    preferred_element_type=jnp.float32, precision=_PREC))
    g2_ref[0] = dinv[:, None] * h2


def _mm2(s1, g1, degp, b1r, xp, W2a, W2b_pad, blk):
    grid = (N // blk, 2)
    return pl.pallas_call(
        _mm2_body,
        grid=grid,
        in_specs=[
            pl.BlockSpec((2, blk, HALF), lambda i, j: (0, i, 0)),
            pl.BlockSpec((2, blk, HALF), lambda i, j: (0, i, 0)),
            pl.BlockSpec((2, blk, HALF), lambda i, j: (0, i, 0)),
            pl.BlockSpec((1, F), lambda i, j: (0, 0)),
            pl.BlockSpec((blk, HALF), lambda i, j: (i, 0)),
            pl.BlockSpec((F, HALF), lambda i, j: (0, j)),
            pl.BlockSpec((HALF, HALF), lambda i, j: (0, j)),
        ],
        out_specs=pl.BlockSpec((1, blk, HALF), lambda i, j: (j, i, 0)),
        out_shape=jax.ShapeDtypeStruct((2, N, HALF), jnp.float32),
    )(s1, g1, degp, b1r, xp, W2a, W2b_pad)


def _final_body(s2_ref, g2_ref, degp_ref, b2_ref, out_ref):
    dinv = _dinv_of(degp_ref[...])
    pre = dinv[None, :, None] * (s2_ref[...] + g2_ref[...])
    out_ref[...] = jnp.concatenate([pre[0], pre[1]], axis=1) + b2_ref[0]


def _final(s2, g2, degp, b2r, blk):
    return pl.pallas_call(
        _final_body,
        grid=(N // blk,),
        in_specs=[
            pl.BlockSpec((2, blk, HALF), lambda i: (0, i, 0)),
            pl.BlockSpec((2, blk, HALF), lambda i: (0, i, 0)),
            pl.BlockSpec((2, blk, HALF), lambda i: (0, i, 0)),
            pl.BlockSpec((1, F), lambda i: (0, 0)),
        ],
        out_specs=pl.BlockSpec((blk, F), lambda i: (i, 0)),
        out_shape=jax.ShapeDtypeStruct((N, F), jnp.float32),
    )(s2, g2, degp, b2r)


def kernel(edge_index, x, Wp, bp, W1, b1, W2, b2):
    blk = 1000
    Wp_pad = jnp.pad(Wp, ((0, 0), (0, HALF - Wp.shape[1])))
    bp_pad = jnp.pad(bp, (0, HALF - bp.shape[0])).reshape(1, HALF)
    W2a = W2[:F]
    W2b_pad = jnp.pad(W2[F:], ((0, HALF - (W2.shape[0] - F)), (0, 0)))
    b1r = b1.reshape(1, F)
    b2r = b2.reshape(1, F)
    ones_v = jnp.ones((CHUNK, HALF), jnp.float32)
    zeros_half = jnp.zeros((RPT, HALF), jnp.float32)

    degp = _deg(edge_index, ones_v, zeros_half)
    g1, xp = _mm1(x, W1, Wp_pad, bp_pad, degp, blk)
    s1 = _agg(edge_index, g1.reshape(2 * N, HALF), zeros_half)
    g2 = _mm2(s1, g1, degp, b1r, xp, W2a, W2b_pad, blk)
    s2 = _agg(edge_index, g2.reshape(2 * N, HALF), zeros_half)
    return _final(s2, g2, degp, b2r, blk)
